# baseline (speedup 1.0000x reference)
import functools

import jax
import jax.numpy as jnp
from jax import lax
from jax.experimental import pallas as pl
from jax.experimental.pallas import tpu as pltpu
from jax.experimental.pallas import tpu_sc as plsc

_BINS = 1024
_MIN_V = -4.0
_MAX_V = 4.0
_N = 8388608
_L = 16
_NC = 2
_NS = 16
_NW = _NC * _NS
_PER_W = _N // _NW
_CHUNK = 32768
_NCHUNK = _PER_W // _CHUNK
_VECS = _CHUNK // _L
_STRIDE = _BINS + 1
_BPT = 128
_NMERGE = _BINS // _BPT

_SCALE = float(_BINS) / (_MAX_V - _MIN_V)
_OFFSET = -_MIN_V * _SCALE

_mesh = plsc.VectorSubcoreMesh(core_axis_name="c", subcore_axis_name="s")


@functools.partial(
    pl.kernel,
    out_type=jax.ShapeDtypeStruct((_NC * _BINS,), jnp.float32),
    mesh=_mesh,
    scratch_types=[
        pltpu.VMEM((_CHUNK,), jnp.float32),
        pltpu.VMEM((_CHUNK,), jnp.float32),
        pltpu.VMEM((_L * _STRIDE,), jnp.float32),
        pltpu.VMEM((_BINS,), jnp.float32),
        pltpu.VMEM_SHARED((_NS * _BINS,), jnp.float32),
        pltpu.VMEM((_NS * _BPT,), jnp.float32),
        pltpu.VMEM((_BPT,), jnp.float32),
        pltpu.SemaphoreType.DMA,
        pltpu.SemaphoreType.DMA,
    ],
    compiler_params=pltpu.CompilerParams(needs_layout_passes=False),
)
def _sc_hist(x_hbm, out_hbm, xbuf0, xbuf1, hlanes, htile, shared, rdbuf,
             obuf, sem0, sem1):
    cid = lax.axis_index("c")
    sid = lax.axis_index("s")
    wid = sid * _NC + cid
    base = wid * _PER_W

    zeros = jnp.zeros((_L,), jnp.float32)
    ones = jnp.ones((_L,), jnp.float32)
    iota = lax.iota(jnp.int32, _L)
    lane_base = iota * _STRIDE
    mx = jnp.full((_L,), _MAX_V, jnp.float32)
    scale = jnp.full((_L,), _SCALE, jnp.float32)
    offs = jnp.full((_L,), _OFFSET, jnp.float32)

    @plsc.parallel_loop(0, (_L * _STRIDE + _L - 1) // _L, unroll=8)
    def _zero(i):
        hlanes[pl.ds(i * _L, _L)] = zeros

    bufs = (xbuf0, xbuf1)
    sems = (sem0, sem1)
    cp = pltpu.async_copy(x_hbm.at[pl.ds(base, _CHUNK)], bufs[0], sems[0])

    for c in range(_NCHUNK):
        cur = c % 2
        if c + 1 < _NCHUNK:
            nxt = pltpu.async_copy(
                x_hbm.at[pl.ds(base + (c + 1) * _CHUNK, _CHUNK)],
                bufs[1 - cur], sems[1 - cur])
        cp.wait()
        buf = bufs[cur]

        @plsc.parallel_loop(0, _VECS, unroll=16)
        def _vec(i):
            xv = buf[pl.ds(i * _L, _L)]
            t = xv * scale + offs
            idx = t.astype(jnp.int32)
            mask = jnp.abs(xv) <= mx
            plsc.addupdate_scatter(hlanes, [lane_base + idx], ones,
                                   mask=mask)

        if c + 1 < _NCHUNK:
            cp = nxt

    ovf = plsc.load_gather(hlanes, [lane_base + _BINS])
    plsc.addupdate_scatter(hlanes, [lane_base + (_BINS - 1)], ovf)

    def _reduce(b, _):
        bins = b * _L + iota
        acc = zeros
        for lane in range(_L):
            acc = acc + plsc.load_gather(hlanes, [lane * _STRIDE + bins])
        htile[pl.ds(b * _L, _L)] = acc
        return ()

    lax.fori_loop(0, _BINS // _L, _reduce, ())

    pltpu.sync_copy(htile, shared.at[pl.ds(sid * _BINS, _BINS)])
    plsc.subcore_barrier()

    @pl.when(sid < _NMERGE)
    def _merge():
        col = pl.multiple_of(sid * _BPT, _BPT)
        for r in range(_NS):
            pltpu.sync_copy(shared.at[pl.ds(r * _BINS + col, _BPT)],
                            rdbuf.at[pl.ds(r * _BPT, _BPT)])

        def _final(v, _):
            acc = zeros
            for r in range(_NS):
                acc = acc + rdbuf[pl.ds(r * _BPT + v * _L, _L)]
            obuf[pl.ds(v * _L, _L)] = acc
            return ()

        lax.fori_loop(0, _BPT // _L, _final, ())
        pltpu.sync_copy(obuf, out_hbm.at[pl.ds(cid * _BINS + col, _BPT)])


def _combine_body(p_ref, o_ref):
    o_ref[...] = p_ref[0] + p_ref[1]


def kernel(x):
    partials = _sc_hist(x).reshape(_NC, _BINS)
    return pl.pallas_call(
        _combine_body,
        out_shape=jax.ShapeDtypeStruct((_BINS,), jnp.float32),
    )(partials)


# speedup vs baseline: 1.0353x; 1.0353x over previous
import functools

import jax
import jax.numpy as jnp
from jax import lax
from jax.experimental import pallas as pl
from jax.experimental.pallas import tpu as pltpu
from jax.experimental.pallas import tpu_sc as plsc

_BINS = 1024
_MAX_V = 4.0
_N = 8388608
_L = 16
_NC = 2
_NS = 16
_NW = _NC * _NS
_PER_W = _N // _NW
_CHUNK = 16384
_NBUF = 4
_NCHUNK = _PER_W // _CHUNK
_VECS = _CHUNK // _L
_STRIDE = _BINS + 1

_SCALE = 128.0
_OFFSET = 512.0

_mesh = plsc.VectorSubcoreMesh(core_axis_name="c", subcore_axis_name="s")


@functools.partial(
    pl.kernel,
    out_type=jax.ShapeDtypeStruct((_NW * _BINS,), jnp.float32),
    mesh=_mesh,
    scratch_types=[
        [pltpu.VMEM((_CHUNK,), jnp.float32) for _ in range(_NBUF)],
        pltpu.VMEM((_L * _STRIDE,), jnp.float32),
        pltpu.VMEM((_BINS,), jnp.float32),
        [pltpu.SemaphoreType.DMA for _ in range(_NBUF)],
    ],
    compiler_params=pltpu.CompilerParams(needs_layout_passes=False),
)
def _sc_hist(x_hbm, out_hbm, bufs, hlanes, htile, sems):
    cid = lax.axis_index("c")
    sid = lax.axis_index("s")
    wid = sid * _NC + cid
    base = wid * _PER_W

    zeros = jnp.zeros((_L,), jnp.float32)
    ones = jnp.ones((_L,), jnp.float32)
    iota = lax.iota(jnp.int32, _L)
    lane_base = iota * _STRIDE
    mx = jnp.full((_L,), _MAX_V, jnp.float32)
    scale = jnp.full((_L,), _SCALE, jnp.float32)
    offs = jnp.full((_L,), _OFFSET, jnp.float32)

    @plsc.parallel_loop(0, _L * _STRIDE // _L, unroll=8)
    def _zero(i):
        hlanes[pl.ds(i * _L, _L)] = zeros

    cps = [pltpu.async_copy(x_hbm.at[pl.ds(base + c * _CHUNK, _CHUNK)],
                            bufs[c], sems[c])
           for c in range(_NBUF)]

    for c in range(_NCHUNK):
        slot = c % _NBUF
        cps[slot].wait()
        buf = bufs[slot]

        @plsc.parallel_loop(0, _VECS, unroll=8)
        def _vec(i):
            xv = buf[pl.ds(i * _L, _L)]
            t = xv * scale + offs
            idx = t.astype(jnp.int32)
            mask = jnp.abs(xv) <= mx
            plsc.addupdate_scatter(hlanes, [lane_base + idx], ones,
                                   mask=mask)

        if c + _NBUF < _NCHUNK:
            cps[slot] = pltpu.async_copy(
                x_hbm.at[pl.ds(base + (c + _NBUF) * _CHUNK, _CHUNK)],
                bufs[slot], sems[slot])

    ovf = plsc.load_gather(hlanes, [lane_base + _BINS])
    plsc.addupdate_scatter(hlanes, [lane_base + (_BINS - 1)], ovf)

    def _reduce(b, _):
        bins = b * _L + iota
        acc = zeros
        for lane in range(_L):
            acc = acc + plsc.load_gather(hlanes, [lane * _STRIDE + bins])
        htile[pl.ds(b * _L, _L)] = acc
        return ()

    lax.fori_loop(0, _BINS // _L, _reduce, ())
    pltpu.sync_copy(htile, out_hbm.at[pl.ds(wid * _BINS, _BINS)])


def _combine_body(p_ref, o_ref):
    o_ref[...] = jnp.sum(p_ref[...], axis=0)


def kernel(x):
    partials = _sc_hist(x).reshape(_NW, _BINS)
    return pl.pallas_call(
        _combine_body,
        out_shape=jax.ShapeDtypeStruct((_BINS,), jnp.float32),
    )(partials)
